# baseline (speedup 1.0000x reference)
import functools

import jax
import jax.numpy as jnp
from jax import lax
from jax.experimental import pallas as pl
from jax.experimental.pallas import tpu as pltpu
from jax.experimental.pallas import tpu_sc as plsc

N = 10000
E = 320000
D_IN = 128
D_HID = 128
D_OUT = 64

NC = 2
NS = 16
NW = NC * NS
EPW = E // NW
CHUNK = 128
NCH = EPW // CHUNK
TAIL = EPW - NCH * CHUNK
CHD = 80
NCHD = EPW // CHD
RPT = 640
RPT_LAST = N - (NS - 1) * RPT
NB = 3
NBI = 6

BN = 2000


def _mesh():
    return plsc.VectorSubcoreMesh(core_axis_name="c", subcore_axis_name="s")


_SC_PARAMS = pltpu.CompilerParams(use_tc_tiling_on_sc=False)


def _stripe_copy(sid, src, dst):
    s0 = sid * RPT

    @pl.when(sid < NS - 1)
    def _():
        pltpu.sync_copy(src.at[pl.ds(s0, RPT)], dst.at[pl.ds(s0, RPT)])

    @pl.when(sid == NS - 1)
    def _():
        pltpu.sync_copy(src.at[pl.ds(s0, RPT_LAST)],
                        dst.at[pl.ds(s0, RPT_LAST)])


def _sc_degree(er, ones, z8):
    @functools.partial(
        pl.kernel,
        out_type=jax.ShapeDtypeStruct((NC, N, 8), jnp.float32),
        mesh=_mesh(),
        scratch_types=[
            pltpu.VMEM((CHD, 8), jnp.float32),
            pltpu.VMEM_SHARED((N, 8), jnp.float32),
            [pltpu.VMEM((CHD,), jnp.int32)] * 8,
            [pltpu.SemaphoreType.DMA] * 8,
            [pltpu.SemaphoreType.DMA] * 8,
        ],
        compiler_params=_SC_PARAMS,
    )
    def deg_kernel(er_hbm, ones_hbm, z_hbm, out_hbm, ones_v, acc,
                   ibufs, isems, ssems):
        core = lax.axis_index("c")
        sid = lax.axis_index("s")
        wid = core * NS + sid
        _stripe_copy(sid, z_hbm, acc)
        pltpu.sync_copy(ones_hbm, ones_v)
        plsc.subcore_barrier()

        def iload(c, j, w=False):
            cp = (pltpu.make_async_copy if w else pltpu.async_copy)(
                er_hbm.at[0, wid, pl.ds(c * CHD, CHD)], ibufs[j], isems[j])
            if w:
                cp.wait()

        def scat(j):
            pltpu.async_copy(ones_v, acc.at[ibufs[j]], ssems[j], add=True)

        def scat_wait(j):
            pltpu.make_async_copy(ones_v, acc.at[ibufs[j]],
                                  ssems[j]).wait()

        def slot(s, m, swait=True, post=True):
            iload(s, m % 8, w=True)
            scat(m % 8)
            if swait:
                scat_wait((m - 6) % 8)
            if post:
                iload(s + 2, (m + 2) % 8)

        iload(0, 0)
        iload(1, 1)
        for s in range(6):
            slot(s, s, swait=False)

        @pl.loop(0, (NCHD - 13) // 8)
        def _(i):
            for k in range(8):
                slot(8 * i + 6 + k, 6 + k)

        for s in range(NCHD - 7, NCHD):
            slot(s, s, post=(s + 2 < NCHD))
        for c in range(NCHD - 6, NCHD):
            scat_wait(c % 8)

        plsc.subcore_barrier()
        _stripe_copy(sid, acc, out_hbm.at[core])

    return deg_kernel(er, ones, z8)


def _sc_spmm(table, er, zeros, d):
    @functools.partial(
        pl.kernel,
        out_type=jax.ShapeDtypeStruct((NC, N, d), jnp.float32),
        mesh=_mesh(),
        scratch_types=[
            pltpu.VMEM_SHARED((N, d), jnp.float32),
            [pltpu.VMEM((CHUNK,), jnp.int32)] * NBI,
            [pltpu.VMEM((CHUNK,), jnp.int32)] * NBI,
            [pltpu.VMEM((CHUNK, d), jnp.float32)] * NB,
            pltpu.VMEM((TAIL,), jnp.int32),
            pltpu.VMEM((TAIL,), jnp.int32),
            [pltpu.SemaphoreType.DMA] * NBI,
            [pltpu.SemaphoreType.DMA] * NBI,
            [pltpu.SemaphoreType.DMA] * NB,
            [pltpu.SemaphoreType.DMA] * NB,
        ],
        compiler_params=_SC_PARAMS,
    )
    def spmm_kernel(tab_hbm, er_hbm, z_hbm, out_hbm,
                    acc, rbufs, cbufs, bufs, rt_v, ct_v,
                    rsems, csems, gsems, ssems):
        core = lax.axis_index("c")
        sid = lax.axis_index("s")
        wid = core * NS + sid
        _stripe_copy(sid, z_hbm, acc)
        plsc.subcore_barrier()

        def iload(c, j, w=False):
            if w:
                pltpu.make_async_copy(er_hbm.at[0, wid, pl.ds(c * CHUNK, CHUNK)],
                                      rbufs[j], rsems[j]).wait()
                pltpu.make_async_copy(er_hbm.at[1, wid, pl.ds(c * CHUNK, CHUNK)],
                                      cbufs[j], csems[j]).wait()
            else:
                pltpu.async_copy(er_hbm.at[0, wid, pl.ds(c * CHUNK, CHUNK)],
                                 rbufs[j], rsems[j])
                pltpu.async_copy(er_hbm.at[1, wid, pl.ds(c * CHUNK, CHUNK)],
                                 cbufs[j], csems[j])

        def gather(j, ji, w=False):
            cp = (pltpu.make_async_copy if w else pltpu.async_copy)(
                tab_hbm.at[rbufs[ji]], bufs[j], gsems[j])
            if w:
                cp.wait()

        def scat(j, ji):
            pltpu.async_copy(bufs[j], acc.at[cbufs[ji]], ssems[j], add=True)

        def scat_wait(j, ji):
            pltpu.make_async_copy(bufs[j], acc.at[cbufs[ji]],
                                  ssems[j]).wait()

        def slot(s, m, swait=True, pre=True, post=True):
            gather(m % NB, m % NBI, w=True)
            scat(m % NB, m % NBI)
            if swait:
                scat_wait((m - 1) % NB, (m - 1) % NBI)
            if pre:
                iload(s + 2, (m + 2) % NBI, w=True)
                gather((m + 2) % NB, (m + 2) % NBI)
            if post:
                iload(s + 4, (m + 4) % NBI)

        for c in range(4):
            iload(c, c)
        for c in range(2):
            iload(c, c, w=True)
            gather(c, c)
        slot(0, 0, swait=False)

        @pl.loop(0, (NCH - 6) // NBI)
        def _(i):
            for k in range(NBI):
                slot(NBI * i + 1 + k, 1 + k)

        for s in range(NCH - 5, NCH):
            slot(s, s, pre=(s + 2 < NCH), post=(s + 4 < NCH))
        scat_wait((NCH - 1) % NB, (NCH - 1) % NBI)

        pltpu.sync_copy(er_hbm.at[0, wid, pl.ds(NCH * CHUNK, TAIL)], rt_v)
        pltpu.sync_copy(er_hbm.at[1, wid, pl.ds(NCH * CHUNK, TAIL)], ct_v)
        pltpu.sync_copy(tab_hbm.at[rt_v], bufs[0].at[pl.ds(0, TAIL)])
        pltpu.sync_copy(bufs[0].at[pl.ds(0, TAIL)], acc.at[ct_v], add=True)

        plsc.subcore_barrier()
        _stripe_copy(sid, acc, out_hbm.at[core])

    return spmm_kernel(table, er, zeros)


def _dis(dref):
    return lax.rsqrt(dref[0, :, :1] + dref[1, :, :1] + 1.0)


def _deg_spec():
    return pl.BlockSpec((NC, BN, 8), lambda i: (0, i, 0))


def _tc_scale(x, deg):
    def body(x_ref, d_ref, xs_ref):
        xs_ref[...] = x_ref[...] * _dis(d_ref)

    return pl.pallas_call(
        body,
        out_shape=jax.ShapeDtypeStruct((N, D_IN), jnp.float32),
        grid=(N // BN,),
        in_specs=[
            pl.BlockSpec((BN, D_IN), lambda i: (i, 0)),
            _deg_spec(),
        ],
        out_specs=pl.BlockSpec((BN, D_IN), lambda i: (i, 0)),
    )(x, deg)


def _tc_layer1(ag, xs, deg, W1, b1, W2):
    def body(a_ref, xs_ref, d_ref, w1, b1r, w2, g_ref):
        dis = _dis(d_ref)
        tot = (a_ref[0] + a_ref[1] + xs_ref[...]) * dis
        h = lax.dot_general(tot, w1[...], (((1,), (1,)), ((), ())),
                            preferred_element_type=jnp.float32)
        h = jnp.maximum(h + b1r[...], 0.0)
        g = lax.dot_general(h, w2[...], (((1,), (1,)), ((), ())),
                            preferred_element_type=jnp.float32)
        g_ref[...] = g * dis

    return pl.pallas_call(
        body,
        out_shape=jax.ShapeDtypeStruct((N, D_OUT), jnp.float32),
        grid=(N // BN,),
        in_specs=[
            pl.BlockSpec((NC, BN, D_HID), lambda i: (0, i, 0)),
            pl.BlockSpec((BN, D_IN), lambda i: (i, 0)),
            _deg_spec(),
            pl.BlockSpec((D_HID, D_IN), lambda i: (0, 0)),
            pl.BlockSpec((1, D_HID), lambda i: (0, 0)),
            pl.BlockSpec((D_OUT, D_HID), lambda i: (0, 0)),
        ],
        out_specs=pl.BlockSpec((BN, D_OUT), lambda i: (i, 0)),
    )(ag, xs, deg, W1, b1, W2)


def _tc_out(ag, g, deg, b2):
    def body(a_ref, g_ref, d_ref, b2r, o_ref):
        dis = _dis(d_ref)
        z = (a_ref[0] + a_ref[1] + g_ref[...]) * dis + b2r[...]
        m = jnp.max(z, axis=1, keepdims=True)
        lse = jnp.log(jnp.sum(jnp.exp(z - m), axis=1, keepdims=True)) + m
        o_ref[...] = z - lse

    return pl.pallas_call(
        body,
        out_shape=jax.ShapeDtypeStruct((N, D_OUT), jnp.float32),
        grid=(N // BN,),
        in_specs=[
            pl.BlockSpec((NC, BN, D_OUT), lambda i: (0, i, 0)),
            pl.BlockSpec((BN, D_OUT), lambda i: (i, 0)),
            _deg_spec(),
            pl.BlockSpec((1, D_OUT), lambda i: (0, 0)),
        ],
        out_specs=pl.BlockSpec((BN, D_OUT), lambda i: (i, 0)),
    )(ag, g, deg, b2)


def kernel(x, edge_index, W1, b1, W2, b2):
    er = edge_index.astype(jnp.int32).reshape(2, NW, EPW)
    ones = jnp.ones((CHD, 8), jnp.float32)
    z8 = jnp.zeros((N, 8), jnp.float32)
    z128 = jnp.zeros((N, D_IN), jnp.float32)
    z64 = jnp.zeros((N, D_OUT), jnp.float32)

    deg = _sc_degree(er, ones, z8)
    xs = _tc_scale(x, deg)
    ag1 = _sc_spmm(xs, er, z128, D_IN)
    g = _tc_layer1(ag1, xs, deg, W1, b1.reshape(1, D_HID), W2)
    ag2 = _sc_spmm(g, er, z64, D_OUT)
    return _tc_out(ag2, g, deg, b2.reshape(1, D_OUT))


# speedup vs baseline: 1.0781x; 1.0781x over previous
import functools

import jax
import jax.numpy as jnp
from jax import lax
from jax.experimental import pallas as pl
from jax.experimental.pallas import tpu as pltpu
from jax.experimental.pallas import tpu_sc as plsc

N = 10000
E = 320000
D_IN = 128
D_HID = 128
D_OUT = 64

NC = 2
NS = 16
NW = NC * NS
EPW = E // NW
CHUNK = 128
NCH = EPW // CHUNK
TAIL = EPW - NCH * CHUNK
CHD = 80
NCHD = EPW // CHD
RPT = 640
RPT_LAST = N - (NS - 1) * RPT
NB = 3
NBI = 6

BN = 2000


def _mesh():
    return plsc.VectorSubcoreMesh(core_axis_name="c", subcore_axis_name="s")


_SC_PARAMS = pltpu.CompilerParams(use_tc_tiling_on_sc=False)


def _stripe_copy(sid, src, dst):
    s0 = sid * RPT

    @pl.when(sid < NS - 1)
    def _():
        pltpu.sync_copy(src.at[pl.ds(s0, RPT)], dst.at[pl.ds(s0, RPT)])

    @pl.when(sid == NS - 1)
    def _():
        pltpu.sync_copy(src.at[pl.ds(s0, RPT_LAST)],
                        dst.at[pl.ds(s0, RPT_LAST)])


def _sc_degree(er, ones, z8):
    @functools.partial(
        pl.kernel,
        out_type=jax.ShapeDtypeStruct((NC, N, 8), jnp.float32),
        mesh=_mesh(),
        scratch_types=[
            pltpu.VMEM((CHD, 8), jnp.float32),
            pltpu.VMEM_SHARED((N, 8), jnp.float32),
            [pltpu.VMEM((CHD,), jnp.int32)] * 12,
            [pltpu.SemaphoreType.DMA] * 12,
            [pltpu.SemaphoreType.DMA] * 12,
        ],
        compiler_params=_SC_PARAMS,
    )
    def deg_kernel(er_hbm, ones_hbm, z_hbm, out_hbm, ones_v, acc,
                   ibufs, isems, ssems):
        core = lax.axis_index("c")
        sid = lax.axis_index("s")
        wid = core * NS + sid
        _stripe_copy(sid, z_hbm, acc)
        pltpu.sync_copy(ones_hbm, ones_v)
        plsc.subcore_barrier()

        def iload(c, j, w=False):
            cp = (pltpu.make_async_copy if w else pltpu.async_copy)(
                er_hbm.at[0, wid, pl.ds(c * CHD, CHD)], ibufs[j], isems[j])
            if w:
                cp.wait()

        def scat(j):
            pltpu.async_copy(ones_v, acc.at[ibufs[j]], ssems[j], add=True)

        def scat_wait(j):
            pltpu.make_async_copy(ones_v, acc.at[ibufs[j]],
                                  ssems[j]).wait()

        def slot(s, m, swait=True, post=True):
            iload(s, m % 12, w=True)
            scat(m % 12)
            if swait:
                scat_wait((m - 2) % 12)
            if post:
                iload(s + 10, (m + 10) % 12)

        for c in range(10):
            iload(c, c)
        for s in range(2):
            slot(s, s, swait=False)

        @pl.loop(0, 9)
        def _(i):
            for k in range(12):
                slot(12 * i + 2 + k, 2 + k)

        for s in range(110, NCHD):
            slot(s, s, post=(s + 10 < NCHD))
        scat_wait((NCHD - 2) % 12)
        scat_wait((NCHD - 1) % 12)

        plsc.subcore_barrier()
        _stripe_copy(sid, acc, out_hbm.at[core])

    return deg_kernel(er, ones, z8)


def _sc_spmm(table, er, zeros, d):
    @functools.partial(
        pl.kernel,
        out_type=jax.ShapeDtypeStruct((NC, N, d), jnp.float32),
        mesh=_mesh(),
        scratch_types=[
            pltpu.VMEM_SHARED((N, d), jnp.float32),
            [pltpu.VMEM((CHUNK,), jnp.int32)] * NBI,
            [pltpu.VMEM((CHUNK,), jnp.int32)] * NBI,
            [pltpu.VMEM((CHUNK, d), jnp.float32)] * NB,
            pltpu.VMEM((TAIL,), jnp.int32),
            pltpu.VMEM((TAIL,), jnp.int32),
            [pltpu.SemaphoreType.DMA] * NBI,
            [pltpu.SemaphoreType.DMA] * NBI,
            [pltpu.SemaphoreType.DMA] * NB,
            [pltpu.SemaphoreType.DMA] * NB,
        ],
        compiler_params=_SC_PARAMS,
    )
    def spmm_kernel(tab_hbm, er_hbm, z_hbm, out_hbm,
                    acc, rbufs, cbufs, bufs, rt_v, ct_v,
                    rsems, csems, gsems, ssems):
        core = lax.axis_index("c")
        sid = lax.axis_index("s")
        wid = core * NS + sid
        _stripe_copy(sid, z_hbm, acc)
        plsc.subcore_barrier()

        def iload(c, j, w=False):
            if w:
                pltpu.make_async_copy(er_hbm.at[0, wid, pl.ds(c * CHUNK, CHUNK)],
                                      rbufs[j], rsems[j]).wait()
                pltpu.make_async_copy(er_hbm.at[1, wid, pl.ds(c * CHUNK, CHUNK)],
                                      cbufs[j], csems[j]).wait()
            else:
                pltpu.async_copy(er_hbm.at[0, wid, pl.ds(c * CHUNK, CHUNK)],
                                 rbufs[j], rsems[j])
                pltpu.async_copy(er_hbm.at[1, wid, pl.ds(c * CHUNK, CHUNK)],
                                 cbufs[j], csems[j])

        def gather(j, ji, w=False):
            cp = (pltpu.make_async_copy if w else pltpu.async_copy)(
                tab_hbm.at[rbufs[ji]], bufs[j], gsems[j])
            if w:
                cp.wait()

        def scat(j, ji):
            pltpu.async_copy(bufs[j], acc.at[cbufs[ji]], ssems[j], add=True)

        def scat_wait(j, ji):
            pltpu.make_async_copy(bufs[j], acc.at[cbufs[ji]],
                                  ssems[j]).wait()

        def slot(s, m, swait=True, pre=True, post=True):
            gather(m % NB, m % NBI, w=True)
            scat(m % NB, m % NBI)
            if swait:
                scat_wait((m - 1) % NB, (m - 1) % NBI)
            if pre:
                iload(s + 2, (m + 2) % NBI, w=True)
                gather((m + 2) % NB, (m + 2) % NBI)
            if post:
                iload(s + 4, (m + 4) % NBI)

        for c in range(4):
            iload(c, c)
        for c in range(2):
            iload(c, c, w=True)
            gather(c, c)
        slot(0, 0, swait=False)

        @pl.loop(0, (NCH - 6) // NBI)
        def _(i):
            for k in range(NBI):
                slot(NBI * i + 1 + k, 1 + k)

        for s in range(NCH - 5, NCH):
            slot(s, s, pre=(s + 2 < NCH), post=(s + 4 < NCH))
        scat_wait((NCH - 1) % NB, (NCH - 1) % NBI)

        pltpu.sync_copy(er_hbm.at[0, wid, pl.ds(NCH * CHUNK, TAIL)], rt_v)
        pltpu.sync_copy(er_hbm.at[1, wid, pl.ds(NCH * CHUNK, TAIL)], ct_v)
        pltpu.sync_copy(tab_hbm.at[rt_v], bufs[0].at[pl.ds(0, TAIL)])
        pltpu.sync_copy(bufs[0].at[pl.ds(0, TAIL)], acc.at[ct_v], add=True)

        plsc.subcore_barrier()
        _stripe_copy(sid, acc, out_hbm.at[core])

    return spmm_kernel(table, er, zeros)


def _dis(dref):
    return lax.rsqrt(dref[0, :, :1] + dref[1, :, :1] + 1.0)


def _deg_spec():
    return pl.BlockSpec((NC, BN, 8), lambda i: (0, i, 0))


def _tc_scale(x, deg):
    def body(x_ref, d_ref, xs_ref):
        xs_ref[...] = x_ref[...] * _dis(d_ref)

    return pl.pallas_call(
        body,
        out_shape=jax.ShapeDtypeStruct((N, D_IN), jnp.float32),
        grid=(N // BN,),
        in_specs=[
            pl.BlockSpec((BN, D_IN), lambda i: (i, 0)),
            _deg_spec(),
        ],
        out_specs=pl.BlockSpec((BN, D_IN), lambda i: (i, 0)),
    )(x, deg)


def _tc_layer1(ag, xs, deg, W1, b1, W2):
    def body(a_ref, xs_ref, d_ref, w1, b1r, w2, g_ref):
        dis = _dis(d_ref)
        tot = (a_ref[0] + a_ref[1] + xs_ref[...]) * dis
        h = lax.dot_general(tot, w1[...], (((1,), (1,)), ((), ())),
                            preferred_element_type=jnp.float32)
        h = jnp.maximum(h + b1r[...], 0.0)
        g = lax.dot_general(h, w2[...], (((1,), (1,)), ((), ())),
                            preferred_element_type=jnp.float32)
        g_ref[...] = g * dis

    return pl.pallas_call(
        body,
        out_shape=jax.ShapeDtypeStruct((N, D_OUT), jnp.float32),
        grid=(N // BN,),
        in_specs=[
            pl.BlockSpec((NC, BN, D_HID), lambda i: (0, i, 0)),
            pl.BlockSpec((BN, D_IN), lambda i: (i, 0)),
            _deg_spec(),
            pl.BlockSpec((D_HID, D_IN), lambda i: (0, 0)),
            pl.BlockSpec((1, D_HID), lambda i: (0, 0)),
            pl.BlockSpec((D_OUT, D_HID), lambda i: (0, 0)),
        ],
        out_specs=pl.BlockSpec((BN, D_OUT), lambda i: (i, 0)),
    )(ag, xs, deg, W1, b1, W2)


def _tc_out(ag, g, deg, b2):
    def body(a_ref, g_ref, d_ref, b2r, o_ref):
        dis = _dis(d_ref)
        z = (a_ref[0] + a_ref[1] + g_ref[...]) * dis + b2r[...]
        m = jnp.max(z, axis=1, keepdims=True)
        lse = jnp.log(jnp.sum(jnp.exp(z - m), axis=1, keepdims=True)) + m
        o_ref[...] = z - lse

    return pl.pallas_call(
        body,
        out_shape=jax.ShapeDtypeStruct((N, D_OUT), jnp.float32),
        grid=(N // BN,),
        in_specs=[
            pl.BlockSpec((NC, BN, D_OUT), lambda i: (0, i, 0)),
            pl.BlockSpec((BN, D_OUT), lambda i: (i, 0)),
            _deg_spec(),
            pl.BlockSpec((1, D_OUT), lambda i: (0, 0)),
        ],
        out_specs=pl.BlockSpec((BN, D_OUT), lambda i: (i, 0)),
    )(ag, g, deg, b2)


def kernel(x, edge_index, W1, b1, W2, b2):
    er = edge_index.astype(jnp.int32).reshape(2, NW, EPW)
    ones = jnp.ones((CHD, 8), jnp.float32)
    z8 = jnp.zeros((N, 8), jnp.float32)
    z128 = jnp.zeros((N, D_IN), jnp.float32)
    z64 = jnp.zeros((N, D_OUT), jnp.float32)

    deg = _sc_degree(er, ones, z8)
    xs = _tc_scale(x, deg)
    ag1 = _sc_spmm(xs, er, z128, D_IN)
    g = _tc_layer1(ag1, xs, deg, W1, b1.reshape(1, D_HID), W2)
    ag2 = _sc_spmm(g, er, z64, D_OUT)
    return _tc_out(ag2, g, deg, b2.reshape(1, D_OUT))
